# baseline (speedup 1.0000x reference)
import jax
import jax.numpy as jnp
from jax import lax
from jax.experimental import pallas as pl
from jax.experimental.pallas import tpu as pltpu
from jax.experimental.pallas import tpu_sc as plsc

B = 16384
F = 26
V = 1000000
NC = 2
NS = 16
NW = NC * NS
BPW = B // NW
CHUNK = 128
NCH = BPW // CHUNK
NJ = F * NCH
GRP = 13


LCH = 62464
NST = 1
VT = 16 * LCH


def _body(
    xt_hbm, tab_hbm, tail_hbm, bias_hbm, out_hbm,
    idx_v, buf_v, acc_v, bias_v, stg_v, spm, sem,
):
    cid = lax.axis_index("c")
    sid = lax.axis_index("s")
    wid = sid * NC + cid

    cps = []
    for k in range(NST):
        off = pl.multiple_of((k * NS + sid) * LCH, 1024)
        cps.append(
            pltpu.async_copy(
                tab_hbm.at[0, pl.ds(off, LCH)], spm.at[pl.ds(off, LCH)], sem
            )
        )
    cps.append(
        pltpu.async_copy(xt_hbm.at[:, pl.ds(wid * BPW, BPW)], idx_v, sem)
    )
    cps.append(pltpu.async_copy(bias_hbm, bias_v, sem))
    for cp in cps:
        cp.wait()

    @pl.when(sid == 0)
    def _():
        pltpu.sync_copy(tail_hbm, stg_v)
        pltpu.sync_copy(stg_v, spm.at[pl.ds(VT, V - VT)])

    binit = bias_v[...]

    plsc.subcore_barrier()

    for g in range(BPW // 16):
        acc_v[pl.ds(g * 16, 16)] = binit

    def fire(f):
        return [
            pltpu.async_copy(
                spm.at[idx_v.at[f, pl.ds(c * CHUNK, CHUNK)]],
                buf_v.at[f, pl.ds(c * CHUNK, CHUNK)],
                sem,
            )
            for c in range(NCH)
        ]

    def fold(f):
        for g in range(BPW // 16):
            s = pl.ds(g * 16, 16)
            acc_v[s] = acc_v[s] + buf_v[f, s]

    DEPTH = 3
    pend = [fire(f) for f in range(DEPTH)]
    for f in range(F):
        if f + DEPTH < F:
            pend.append(fire(f + DEPTH))
        for cp in pend.pop(0):
            cp.wait()
        fold(f)

    pltpu.sync_copy(acc_v, out_hbm.at[pl.ds(wid * BPW, BPW)])


@jax.jit
def _linear_features(xt, tab, tail, bias):
    mesh = plsc.VectorSubcoreMesh(core_axis_name="c", subcore_axis_name="s")
    return pl.kernel(
        _body,
        out_type=jax.ShapeDtypeStruct((B,), jnp.float32),
        mesh=mesh,
        compiler_params=pltpu.CompilerParams(needs_layout_passes=False),
        scratch_types=[
            pltpu.VMEM((F, BPW), jnp.int32),
            pltpu.VMEM((F, BPW), jnp.float32),
            pltpu.VMEM((BPW,), jnp.float32),
            pltpu.VMEM((16,), jnp.float32),
            pltpu.VMEM((V - VT,), jnp.float32),
            pltpu.VMEM_SHARED((V,), jnp.float32),
            pltpu.SemaphoreType.DMA,
        ],
    )(xt, tab, tail, bias)


def kernel(x, fc_weight, bias):
    out = _linear_features(
        x.astype(jnp.int32).T,
        fc_weight.T,
        fc_weight[VT:].reshape(-1),
        jnp.broadcast_to(bias, (16,)),
    )
    return out.reshape(B, 1)


# speedup vs baseline: 1.0306x; 1.0306x over previous
import jax
import jax.numpy as jnp
from jax import lax
from jax.experimental import pallas as pl
from jax.experimental.pallas import tpu as pltpu
from jax.experimental.pallas import tpu_sc as plsc

B = 16384
F = 26
V = 1000000
NC = 2
NS = 16
NW = NC * NS
BPW = B // NW
CHUNK = 128
NCH = BPW // CHUNK
NJ = F * NCH
GRP = 13


LCH = 62464
NST = 1
VT = 16 * LCH


def _body(
    xt_hbm, tab_hbm, tail_hbm, bias_hbm, out_hbm,
    idx_v, buf_v, acc_v, bias_v, stg_v, spm, sem,
):
    cid = lax.axis_index("c")
    sid = lax.axis_index("s")
    wid = sid * NC + cid

    cps = []
    for k in range(NST):
        off = pl.multiple_of((k * NS + sid) * LCH, 1024)
        cps.append(
            pltpu.async_copy(
                tab_hbm.at[0, pl.ds(off, LCH)], spm.at[pl.ds(off, LCH)], sem
            )
        )
    cps.append(
        pltpu.async_copy(xt_hbm.at[:, pl.ds(wid * BPW, BPW)], idx_v, sem)
    )
    cps.append(pltpu.async_copy(bias_hbm, bias_v, sem))
    for cp in cps:
        cp.wait()

    @pl.when(sid == 0)
    def _():
        pltpu.sync_copy(tail_hbm, stg_v)
        pltpu.sync_copy(stg_v, spm.at[pl.ds(VT, V - VT)])

    binit = bias_v[...]

    plsc.subcore_barrier()

    def fire(g):
        cps = []
        for jj in range(GRP):
            j = g * GRP + jj
            f, c = j // NCH, j % NCH
            cps.append(
                pltpu.async_copy(
                    spm.at[idx_v.at[f, pl.ds(c * CHUNK, CHUNK)]],
                    buf_v.at[f, pl.ds(c * CHUNK, CHUNK)],
                    sem,
                )
            )
        return cps

    prev = None
    for g in range(NJ // GRP):
        cur = fire(g)
        if prev is not None:
            for cp in prev:
                cp.wait()
        prev = cur
    for cp in prev:
        cp.wait()

    for g in range(BPW // 16):
        acc16 = binit
        for f in range(F):
            acc16 = acc16 + buf_v[f, pl.ds(g * 16, 16)]
        acc_v[pl.ds(g * 16, 16)] = acc16

    pltpu.sync_copy(acc_v, out_hbm.at[pl.ds(wid * BPW, BPW)])


@jax.jit
def _linear_features(xt, tab, tail, bias):
    mesh = plsc.VectorSubcoreMesh(core_axis_name="c", subcore_axis_name="s")
    return pl.kernel(
        _body,
        out_type=jax.ShapeDtypeStruct((B,), jnp.float32),
        mesh=mesh,
        compiler_params=pltpu.CompilerParams(needs_layout_passes=False),
        scratch_types=[
            pltpu.VMEM((F, BPW), jnp.int32),
            pltpu.VMEM((F, BPW), jnp.float32),
            pltpu.VMEM((BPW,), jnp.float32),
            pltpu.VMEM((16,), jnp.float32),
            pltpu.VMEM((V - VT,), jnp.float32),
            pltpu.VMEM_SHARED((V,), jnp.float32),
            pltpu.SemaphoreType.DMA,
        ],
    )(xt, tab, tail, bias)


def kernel(x, fc_weight, bias):
    out = _linear_features(
        x.astype(jnp.int32).T,
        fc_weight.T,
        fc_weight[VT:].reshape(-1),
        jnp.broadcast_to(bias, (16,)),
    )
    return out.reshape(B, 1)


# speedup vs baseline: 1.0384x; 1.0076x over previous
import jax
import jax.numpy as jnp
from jax import lax
from jax.experimental import pallas as pl
from jax.experimental.pallas import tpu as pltpu
from jax.experimental.pallas import tpu_sc as plsc

B = 16384
F = 26
V = 1000000
NC = 2
NS = 16
NW = NC * NS
BPW = B // NW
CHUNK = 128
NCH = BPW // CHUNK
NJ = F * NCH
GRP = 26


LCH = 62464
NST = 1
VT = 16 * LCH


def _body(
    xt_hbm, tab_hbm, tail_hbm, bias_hbm, out_hbm,
    idx_v, buf_v, acc_v, bias_v, stg_v, spm, sem,
):
    cid = lax.axis_index("c")
    sid = lax.axis_index("s")
    wid = sid * NC + cid

    cps = []
    for k in range(NST):
        off = pl.multiple_of((k * NS + sid) * LCH, 1024)
        cps.append(
            pltpu.async_copy(
                tab_hbm.at[0, pl.ds(off, LCH)], spm.at[pl.ds(off, LCH)], sem
            )
        )
    cps.append(
        pltpu.async_copy(xt_hbm.at[:, pl.ds(wid * BPW, BPW)], idx_v, sem)
    )
    cps.append(pltpu.async_copy(bias_hbm, bias_v, sem))
    for cp in cps:
        cp.wait()

    @pl.when(sid == 0)
    def _():
        pltpu.sync_copy(tail_hbm, stg_v)
        pltpu.sync_copy(stg_v, spm.at[pl.ds(VT, V - VT)])

    binit = bias_v[...]

    plsc.subcore_barrier()

    def fire(g):
        cps = []
        for jj in range(GRP):
            j = g * GRP + jj
            f, c = j // NCH, j % NCH
            cps.append(
                pltpu.async_copy(
                    spm.at[idx_v.at[f, pl.ds(c * CHUNK, CHUNK)]],
                    buf_v.at[f, pl.ds(c * CHUNK, CHUNK)],
                    sem,
                )
            )
        return cps

    prev = None
    for g in range(NJ // GRP):
        cur = fire(g)
        if prev is not None:
            for cp in prev:
                cp.wait()
        prev = cur
    for cp in prev:
        cp.wait()

    for g in range(BPW // 16):
        acc16 = binit
        for f in range(F):
            acc16 = acc16 + buf_v[f, pl.ds(g * 16, 16)]
        acc_v[pl.ds(g * 16, 16)] = acc16

    pltpu.sync_copy(acc_v, out_hbm.at[pl.ds(wid * BPW, BPW)])


@jax.jit
def _linear_features(xt, tab, tail, bias):
    mesh = plsc.VectorSubcoreMesh(core_axis_name="c", subcore_axis_name="s")
    return pl.kernel(
        _body,
        out_type=jax.ShapeDtypeStruct((B,), jnp.float32),
        mesh=mesh,
        compiler_params=pltpu.CompilerParams(needs_layout_passes=False),
        scratch_types=[
            pltpu.VMEM((F, BPW), jnp.int32),
            pltpu.VMEM((F, BPW), jnp.float32),
            pltpu.VMEM((BPW,), jnp.float32),
            pltpu.VMEM((16,), jnp.float32),
            pltpu.VMEM((V - VT,), jnp.float32),
            pltpu.VMEM_SHARED((V,), jnp.float32),
            pltpu.SemaphoreType.DMA,
        ],
    )(xt, tab, tail, bias)


def kernel(x, fc_weight, bias):
    out = _linear_features(
        x.astype(jnp.int32).T,
        fc_weight.T,
        fc_weight[VT:].reshape(-1),
        jnp.broadcast_to(bias, (16,)),
    )
    return out.reshape(B, 1)
